# Initial kernel scaffold; baseline (speedup 1.0000x reference)
import functools

import jax
import jax.numpy as jnp
from jax import lax
from jax.experimental import pallas as pl
from jax.experimental.pallas import tpu as pltpu
from jax.experimental.pallas import tpu_sc as plsc

N = 10000
NPAD = 10240
RB = 640
GRID = NPAD // RB
C = 128
NC, NS, L = 2, 16, 16

_f32 = jnp.float32


def _full_spec(shape):
  nd = len(shape)
  return pl.BlockSpec(shape, lambda i: (0,) * nd)


def _row_spec(d):
  return pl.BlockSpec((RB, d), lambda i: (i, 0))


def _enc_body(xphy, xpoi, xse, xsc,
              p1w, p1b, p2w, p2b, o1w, o1b, o2w, o2b,
              s1w, s1b, s2w, s2b, c1w, c1b, c2w, c2b,
              wp, wo, ws, wc, avs, avd,
              h1_ref, asrc_ref, adst_ref, ms_ref, md_ref):
  relu = lambda v: jnp.maximum(v, 0.0)
  xp = relu(relu(xphy[...] @ p1w[...] + p1b[...]) @ p2w[...] + p2b[...])
  xo = relu(relu(xpoi[...] @ o1w[...] + o1b[...]) @ o2w[...] + o2b[...])
  xs = relu(relu(xse[...] @ s1w[...] + s1b[...]) @ s2w[...] + s2b[...])
  z = xsc[...] @ c1w[...] + c1b[...]
  sg = 1.0 / (1.0 + jnp.exp(-z))
  xc = relu(sg @ c2w[...] + c2b[...])
  h1 = xp @ wp[...] + xo @ wo[...] + xs @ ws[...] + xc @ wc[...]
  h1_ref[...] = h1
  a_s = jnp.sum(h1 * avs[...], axis=1, keepdims=True)
  a_d = jnp.sum(h1 * avd[...], axis=1, keepdims=True)
  asrc_ref[...] = a_s
  adst_ref[...] = a_d
  bs = jnp.max(a_s)
  bd = jnp.max(a_d)
  first = pl.program_id(0) == 0

  @pl.when(first)
  def _():
    ms_ref[0, 0] = bs
    md_ref[0, 0] = bd

  @pl.when(jnp.logical_not(first))
  def _():
    ms_ref[0, 0] = jnp.maximum(ms_ref[0, 0], bs)
    md_ref[0, 0] = jnp.maximum(md_ref[0, 0], bd)


def _encoder_call(xphy, xpoi, xse, xsc, wdict):
  in_specs = [_row_spec(2), _row_spec(13), _row_spec(40), _row_spec(365)]
  ws = [wdict[k] for k in ('p1w', 'p1b', 'p2w', 'p2b', 'o1w', 'o1b', 'o2w',
                           'o2b', 's1w', 's1b', 's2w', 's2b', 'c1w', 'c1b',
                           'c2w', 'c2b', 'wp', 'wo', 'ws', 'wc', 'avs', 'avd')]
  in_specs += [_full_spec(w.shape) for w in ws]
  out_shapes = (
      jax.ShapeDtypeStruct((NPAD, 128), _f32),
      jax.ShapeDtypeStruct((NPAD, 1), _f32),
      jax.ShapeDtypeStruct((NPAD, 1), _f32),
      jax.ShapeDtypeStruct((1, 1), _f32),
      jax.ShapeDtypeStruct((1, 1), _f32),
  )
  out_specs = (
      _row_spec(128), _row_spec(1), _row_spec(1),
      pl.BlockSpec((1, 1), lambda i: (0, 0)),
      pl.BlockSpec((1, 1), lambda i: (0, 0)),
  )
  return pl.pallas_call(
      _enc_body, grid=(GRID,), in_specs=in_specs, out_specs=out_specs,
      out_shape=out_shapes)(xphy, xpoi, xse, xsc, *ws)


def _mid_body(g0, g1, b1, w2, avs, avd,
              h2_ref, asrc_ref, adst_ref, ms_ref, md_ref):
  g = jnp.maximum(g0[...] + g1[...] + b1[...], 0.0)
  h2 = g @ w2[...]
  h2_ref[...] = h2
  a_s = jnp.sum(h2 * avs[...], axis=1, keepdims=True)
  a_d = jnp.sum(h2 * avd[...], axis=1, keepdims=True)
  asrc_ref[...] = a_s
  adst_ref[...] = a_d
  bs = jnp.max(a_s)
  bd = jnp.max(a_d)
  first = pl.program_id(0) == 0

  @pl.when(first)
  def _():
    ms_ref[0, 0] = bs
    md_ref[0, 0] = bd

  @pl.when(jnp.logical_not(first))
  def _():
    ms_ref[0, 0] = jnp.maximum(ms_ref[0, 0], bs)
    md_ref[0, 0] = jnp.maximum(md_ref[0, 0], bd)


def _mid_call(g0, g1, b1, w2, avs, avd):
  in_specs = [_row_spec(128), _row_spec(128), _full_spec(b1.shape),
              _full_spec(w2.shape), _full_spec(avs.shape),
              _full_spec(avd.shape)]
  out_shapes = (
      jax.ShapeDtypeStruct((NPAD, 64), _f32),
      jax.ShapeDtypeStruct((NPAD, 1), _f32),
      jax.ShapeDtypeStruct((NPAD, 1), _f32),
      jax.ShapeDtypeStruct((1, 1), _f32),
      jax.ShapeDtypeStruct((1, 1), _f32),
  )
  out_specs = (
      _row_spec(64), _row_spec(1), _row_spec(1),
      pl.BlockSpec((1, 1), lambda i: (0, 0)),
      pl.BlockSpec((1, 1), lambda i: (0, 0)),
  )
  return pl.pallas_call(
      _mid_body, grid=(GRID,), in_specs=in_specs, out_specs=out_specs,
      out_shape=out_shapes)(g0, g1, b1, w2, avs, avd)


def _head_body(q0, q1, b2, f1w, f1b, f2w, f2b, f3w, f3b, out_ref):
  relu = lambda v: jnp.maximum(v, 0.0)
  g = relu(q0[...] + q1[...] + b2[...])
  y = relu(g @ f1w[...] + f1b[...])
  y = relu(y @ f2w[...] + f2b[...])
  out_ref[...] = y @ f3w[...] + f3b[...]


def _head_call(q0, q1, b2, f1w, f1b, f2w, f2b, f3w, f3b):
  ws = [b2, f1w, f1b, f2w, f2b, f3w, f3b]
  in_specs = [_row_spec(64), _row_spec(64)] + [_full_spec(w.shape) for w in ws]
  return pl.pallas_call(
      _head_body, grid=(GRID,), in_specs=in_specs,
      out_specs=_row_spec(24),
      out_shape=jax.ShapeDtypeStruct((NPAD, 24), _f32))(q0, q1, *ws)


def _make_gat_sc(D, EPAD):
  T1 = EPAD // NS
  T2 = EPAD // (NC * NS)
  K1 = T1 // C
  K2 = T2 // C
  NR = NPAD // L
  RPT = NPAD // NS

  mesh = plsc.VectorSubcoreMesh(core_axis_name="c", subcore_axis_name="s",
                                num_cores=NC, num_subcores=NS)

  @functools.partial(
      pl.kernel, mesh=mesh,
      out_type=jax.ShapeDtypeStruct((NC, NPAD, D), _f32),
      scratch_types=dict(
          asrc_t=pltpu.VMEM((NPAD,), _f32),
          adst_t=pltpu.VMEM((NPAD,), _f32),
          s_t=pltpu.VMEM((NR, L), _f32),
          mvec_t=pltpu.VMEM((L,), _f32),
          srcb=pltpu.VMEM((C,), jnp.int32),
          dstb=pltpu.VMEM((C,), jnp.int32),
          linb=pltpu.VMEM((NR,), jnp.int32),
          rows=pltpu.VMEM((C, D), _f32),
          coefb=pltpu.VMEM((C,), _f32),
          s_sc=pltpu.VMEM_SHARED((NR, L), _f32),
          out_sc=pltpu.VMEM_SHARED((NPAD, D), _f32),
          sem=pltpu.SemaphoreType.DMA,
      ),
  )
  def gat(h_hbm, asrc_hbm, adst_hbm, mvec_hbm, src_hbm, dst_hbm, linr_hbm,
          zs_hbm, zout_hbm, out_hbm,
          asrc_t, adst_t, s_t, mvec_t, srcb, dstb, linb, rows, coefb,
          s_sc, out_sc, sem):
    cid = lax.axis_index("c")
    sid = lax.axis_index("s")
    wid = sid * NC + cid

    pltpu.sync_copy(asrc_hbm, asrc_t)
    pltpu.sync_copy(adst_hbm, adst_t)
    pltpu.sync_copy(mvec_hbm, mvec_t)
    pltpu.sync_copy(linr_hbm, linb)
    pltpu.sync_copy(zs_hbm, s_t)
    pltpu.sync_copy(zout_hbm.at[pl.ds(sid * RPT, RPT)],
                    out_sc.at[pl.ds(sid * RPT, RPT)])

    @pl.when(sid == 0)
    def _():
      pltpu.sync_copy(zs_hbm, s_sc)

    mv = mvec_t[...]

    def edge_w(i):
      i16 = i * L
      isrc = srcb[pl.ds(i16, L)]
      idst = dstb[pl.ds(i16, L)]
      a1 = plsc.load_gather(asrc_t, [isrc])
      a2 = plsc.load_gather(adst_t, [idst])
      e = a1 + a2
      e = jnp.where(e >= 0.0, e, 0.2 * e)
      w = jnp.exp(e - mv)
      return w, isrc, idst

    def p1_chunk(k, carry):
      base = sid * T1 + k * C
      pltpu.sync_copy(src_hbm.at[pl.ds(base, C)], srcb)
      pltpu.sync_copy(dst_hbm.at[pl.ds(base, C)], dstb)
      for i in range(C // L):
        w, _, idst = edge_w(i)
        hi = lax.shift_right_logical(idst, 4)
        lo = lax.bitwise_and(idst, L - 1)
        plsc.addupdate_scatter(s_t, [hi, lo], w)
      return carry

    lax.fori_loop(0, K1, p1_chunk, 0)
    plsc.subcore_barrier()

    pltpu.sync_copy(s_t, s_sc.at[linb], add=True)
    plsc.subcore_barrier()
    pltpu.sync_copy(s_sc, s_t)

    def p2_chunk(k, carry):
      base = wid * T2 + k * C
      pltpu.sync_copy(src_hbm.at[pl.ds(base, C)], srcb)
      pltpu.sync_copy(dst_hbm.at[pl.ds(base, C)], dstb)
      pltpu.async_copy(h_hbm.at[srcb], rows, sem).wait()
      for i in range(C // L):
        w, _, idst = edge_w(i)
        hi = lax.shift_right_logical(idst, 4)
        lo = lax.bitwise_and(idst, L - 1)
        sden = plsc.load_gather(s_t, [hi, lo])
        coef = w / (sden + 1e-16)
        coefb[pl.ds(i * L, L)] = coef

      def scale_row(r, rcarry):
        cv = jnp.full((L,), coefb[r], _f32)
        for j in range(D // L):
          rows[r, pl.ds(j * L, L)] = rows[r, pl.ds(j * L, L)] * cv
        return rcarry

      lax.fori_loop(0, C, scale_row, 0)
      pltpu.sync_copy(rows, out_sc.at[dstb], add=True)
      return carry

    lax.fori_loop(0, K2, p2_chunk, 0)
    plsc.subcore_barrier()

    pltpu.sync_copy(out_sc.at[pl.ds(sid * RPT, RPT)],
                    out_hbm.at[cid, pl.ds(sid * RPT, RPT)])

  return gat


def kernel(x, edge_index,
           phy1_W, phy1_b, phy2_W, phy2_b, poi1_W, poi1_b, poi2_W, poi2_b,
           se1_W, se1_b, se2_W, se2_b, scene1_W, scene1_b, scene2_W, scene2_b,
           fc1_W, fc1_b, fc2_W, fc2_b, fc3_W, fc3_b,
           conv1_W, conv1_as, conv1_ad, conv1_b,
           conv2_W, conv2_as, conv2_ad, conv2_b):
  n = x.shape[0]
  e = edge_index.shape[1]
  ea = e + n
  epad = ((ea + NC * NS * C - 1) // (NC * NS * C)) * (NC * NS * C)

  xpad = jnp.zeros((NPAD, x.shape[1]), _f32).at[:n].set(x)
  xphy = xpad[:, 0:2]
  xpoi = xpad[:, 2:15]
  xse = xpad[:, 15:55]
  xsc = xpad[:, 55:]

  wdict = dict(
      p1w=phy1_W, p1b=phy1_b.reshape(1, -1),
      p2w=phy2_W, p2b=phy2_b.reshape(1, -1),
      o1w=poi1_W, o1b=poi1_b.reshape(1, -1),
      o2w=poi2_W, o2b=poi2_b.reshape(1, -1),
      s1w=se1_W, s1b=se1_b.reshape(1, -1),
      s2w=se2_W, s2b=se2_b.reshape(1, -1),
      c1w=scene1_W, c1b=scene1_b.reshape(1, -1),
      c2w=scene2_W, c2b=scene2_b.reshape(1, -1),
      wp=conv1_W[0:64], wo=conv1_W[64:128], ws=conv1_W[128:256],
      wc=conv1_W[256:320],
      avs=conv1_as.reshape(1, -1), avd=conv1_ad.reshape(1, -1),
  )
  h1, a1s, a1d, ms1, md1 = _encoder_call(xphy, xpoi, xse, xsc, wdict)

  loop = jnp.arange(n, dtype=edge_index.dtype)
  padv = jnp.full((epad - ea,), n, edge_index.dtype)
  src = jnp.concatenate([edge_index[0], loop, padv])
  dst = jnp.concatenate([edge_index[1], loop, padv])

  linr = jnp.arange(NPAD // L, dtype=jnp.int32)
  zs = jnp.zeros((NPAD // L, L), _f32)

  def mbound(ms, md):
    m = ms[0, 0] + md[0, 0]
    m = jnp.where(m > 0.0, m, 0.2 * m)
    return jnp.full((L,), m, _f32)

  gat1 = _make_gat_sc(128, epad)
  z128 = jnp.zeros((NPAD, 128), _f32)
  agg1 = gat1(h1, a1s.reshape(NPAD), a1d.reshape(NPAD), mbound(ms1, md1),
              src, dst, linr, zs, z128)

  h2, a2s, a2d, ms2, md2 = _mid_call(
      agg1[0], agg1[1], conv1_b.reshape(1, -1), conv2_W,
      conv2_as.reshape(1, -1), conv2_ad.reshape(1, -1))

  gat2 = _make_gat_sc(64, epad)
  z64 = jnp.zeros((NPAD, 64), _f32)
  agg2 = gat2(h2, a2s.reshape(NPAD), a2d.reshape(NPAD), mbound(ms2, md2),
              src, dst, linr, zs, z64)

  out = _head_call(agg2[0], agg2[1], conv2_b.reshape(1, -1),
                   fc1_W, fc1_b.reshape(1, -1), fc2_W, fc2_b.reshape(1, -1),
                   fc3_W, fc3_b.reshape(1, -1))
  return out[:n]


# speedup vs baseline: 15.7692x; 15.7692x over previous
import functools

import jax
import jax.numpy as jnp
from jax import lax
from jax.experimental import pallas as pl
from jax.experimental.pallas import tpu as pltpu
from jax.experimental.pallas import tpu_sc as plsc

N = 10000
NPAD = 10240
RB = 640
GRID = NPAD // RB
C = 128
NC, NS, L = 2, 16, 16

_f32 = jnp.float32


def _full_spec(shape):
  nd = len(shape)
  return pl.BlockSpec(shape, lambda i: (0,) * nd)


def _row_spec(d):
  return pl.BlockSpec((RB, d), lambda i: (i, 0))


def _enc_body(xphy, xpoi, xse, xsc,
              p1w, p1b, p2w, p2b, o1w, o1b, o2w, o2b,
              s1w, s1b, s2w, s2b, c1w, c1b, c2w, c2b,
              wp, wo, ws, wc, avs, avd,
              h1_ref, asrc_ref, adst_ref, ms_ref, md_ref):
  relu = lambda v: jnp.maximum(v, 0.0)
  xp = relu(relu(xphy[...] @ p1w[...] + p1b[...]) @ p2w[...] + p2b[...])
  xo = relu(relu(xpoi[...] @ o1w[...] + o1b[...]) @ o2w[...] + o2b[...])
  xs = relu(relu(xse[...] @ s1w[...] + s1b[...]) @ s2w[...] + s2b[...])
  z = xsc[...] @ c1w[...] + c1b[...]
  sg = 1.0 / (1.0 + jnp.exp(-z))
  xc = relu(sg @ c2w[...] + c2b[...])
  h1 = xp @ wp[...] + xo @ wo[...] + xs @ ws[...] + xc @ wc[...]
  h1_ref[...] = h1
  a_s = jnp.sum(h1 * avs[...], axis=1, keepdims=True)
  a_d = jnp.sum(h1 * avd[...], axis=1, keepdims=True)
  asrc_ref[...] = a_s
  adst_ref[...] = a_d
  bs = jnp.max(a_s, axis=(0, 1), keepdims=True)
  bd = jnp.max(a_d, axis=(0, 1), keepdims=True)
  first = pl.program_id(0) == 0

  @pl.when(first)
  def _():
    ms_ref[...] = bs
    md_ref[...] = bd

  @pl.when(jnp.logical_not(first))
  def _():
    ms_ref[...] = jnp.maximum(ms_ref[...], bs)
    md_ref[...] = jnp.maximum(md_ref[...], bd)


def _encoder_call(xphy, xpoi, xse, xsc, wdict):
  in_specs = [_row_spec(2), _row_spec(13), _row_spec(40), _row_spec(365)]
  ws = [wdict[k] for k in ('p1w', 'p1b', 'p2w', 'p2b', 'o1w', 'o1b', 'o2w',
                           'o2b', 's1w', 's1b', 's2w', 's2b', 'c1w', 'c1b',
                           'c2w', 'c2b', 'wp', 'wo', 'ws', 'wc', 'avs', 'avd')]
  in_specs += [_full_spec(w.shape) for w in ws]
  out_shapes = (
      jax.ShapeDtypeStruct((NPAD, 128), _f32),
      jax.ShapeDtypeStruct((NPAD, 1), _f32),
      jax.ShapeDtypeStruct((NPAD, 1), _f32),
      jax.ShapeDtypeStruct((1, 1), _f32),
      jax.ShapeDtypeStruct((1, 1), _f32),
  )
  out_specs = (
      _row_spec(128), _row_spec(1), _row_spec(1),
      pl.BlockSpec((1, 1), lambda i: (0, 0)),
      pl.BlockSpec((1, 1), lambda i: (0, 0)),
  )
  return pl.pallas_call(
      _enc_body, grid=(GRID,), in_specs=in_specs, out_specs=out_specs,
      out_shape=out_shapes)(xphy, xpoi, xse, xsc, *ws)


def _mid_body(g0, g1, b1, w2, avs, avd,
              h2_ref, asrc_ref, adst_ref, ms_ref, md_ref):
  g = jnp.maximum(g0[...] + g1[...] + b1[...], 0.0)
  h2 = g @ w2[...]
  h2_ref[...] = h2
  a_s = jnp.sum(h2 * avs[...], axis=1, keepdims=True)
  a_d = jnp.sum(h2 * avd[...], axis=1, keepdims=True)
  asrc_ref[...] = a_s
  adst_ref[...] = a_d
  bs = jnp.max(a_s, axis=(0, 1), keepdims=True)
  bd = jnp.max(a_d, axis=(0, 1), keepdims=True)
  first = pl.program_id(0) == 0

  @pl.when(first)
  def _():
    ms_ref[...] = bs
    md_ref[...] = bd

  @pl.when(jnp.logical_not(first))
  def _():
    ms_ref[...] = jnp.maximum(ms_ref[...], bs)
    md_ref[...] = jnp.maximum(md_ref[...], bd)


def _mid_call(g0, g1, b1, w2, avs, avd):
  in_specs = [_row_spec(128), _row_spec(128), _full_spec(b1.shape),
              _full_spec(w2.shape), _full_spec(avs.shape),
              _full_spec(avd.shape)]
  out_shapes = (
      jax.ShapeDtypeStruct((NPAD, 64), _f32),
      jax.ShapeDtypeStruct((NPAD, 1), _f32),
      jax.ShapeDtypeStruct((NPAD, 1), _f32),
      jax.ShapeDtypeStruct((1, 1), _f32),
      jax.ShapeDtypeStruct((1, 1), _f32),
  )
  out_specs = (
      _row_spec(64), _row_spec(1), _row_spec(1),
      pl.BlockSpec((1, 1), lambda i: (0, 0)),
      pl.BlockSpec((1, 1), lambda i: (0, 0)),
  )
  return pl.pallas_call(
      _mid_body, grid=(GRID,), in_specs=in_specs, out_specs=out_specs,
      out_shape=out_shapes)(g0, g1, b1, w2, avs, avd)


def _head_body(q0, q1, b2, f1w, f1b, f2w, f2b, f3w, f3b, out_ref):
  relu = lambda v: jnp.maximum(v, 0.0)
  g = relu(q0[...] + q1[...] + b2[...])
  y = relu(g @ f1w[...] + f1b[...])
  y = relu(y @ f2w[...] + f2b[...])
  out_ref[...] = y @ f3w[...] + f3b[...]


def _head_call(q0, q1, b2, f1w, f1b, f2w, f2b, f3w, f3b):
  ws = [b2, f1w, f1b, f2w, f2b, f3w, f3b]
  in_specs = [_row_spec(64), _row_spec(64)] + [_full_spec(w.shape) for w in ws]
  return pl.pallas_call(
      _head_body, grid=(GRID,), in_specs=in_specs,
      out_specs=_row_spec(24),
      out_shape=jax.ShapeDtypeStruct((NPAD, 24), _f32))(q0, q1, *ws)


def _make_gat_sc(D, EPAD):
  T1 = EPAD // NS
  T2 = EPAD // (NC * NS)
  K1 = T1 // C
  K2 = T2 // C
  NR = NPAD // L
  RPT = NPAD // NS

  mesh = plsc.VectorSubcoreMesh(core_axis_name="c", subcore_axis_name="s",
                                num_cores=NC, num_subcores=NS)

  @functools.partial(
      pl.kernel, mesh=mesh,
      compiler_params=pltpu.CompilerParams(needs_layout_passes=False,
                                           use_tc_tiling_on_sc=False),
      out_type=jax.ShapeDtypeStruct((NC, NPAD, D), _f32),
      scratch_types=dict(
          asrc_t=pltpu.VMEM((NPAD,), _f32),
          adst_t=pltpu.VMEM((NPAD,), _f32),
          s_t=pltpu.VMEM((NR, L), _f32),
          mvec_t=pltpu.VMEM((L,), _f32),
          srcb=pltpu.VMEM((C,), jnp.int32),
          dstb=pltpu.VMEM((C,), jnp.int32),
          linb=pltpu.VMEM((NR,), jnp.int32),
          rows=pltpu.VMEM((C, D), _f32),
          s_sc=pltpu.VMEM_SHARED((NR, L), _f32),
          out_sc=pltpu.VMEM_SHARED((NPAD, D), _f32),
          sem=pltpu.SemaphoreType.DMA,
      ),
  )
  def gat(h_hbm, asrc_hbm, adst_hbm, mvec_hbm, src_hbm, dst_hbm, linr_hbm,
          zs_hbm, zout_hbm, out_hbm,
          asrc_t, adst_t, s_t, mvec_t, srcb, dstb, linb, rows,
          s_sc, out_sc, sem):
    cid = lax.axis_index("c")
    sid = lax.axis_index("s")
    wid = sid * NC + cid

    pltpu.sync_copy(asrc_hbm, asrc_t)
    pltpu.sync_copy(adst_hbm, adst_t)
    pltpu.sync_copy(mvec_hbm, mvec_t)
    pltpu.sync_copy(linr_hbm, linb)
    pltpu.sync_copy(zs_hbm, s_t)
    pltpu.sync_copy(zout_hbm.at[pl.ds(sid * RPT, RPT)],
                    out_sc.at[pl.ds(sid * RPT, RPT)])

    @pl.when(sid == 0)
    def _():
      pltpu.sync_copy(zs_hbm, s_sc)

    mv = mvec_t[...]

    def edge_w(i):
      i16 = i * L
      isrc = srcb[pl.ds(i16, L)]
      idst = dstb[pl.ds(i16, L)]
      a1 = plsc.load_gather(asrc_t, [isrc])
      a2 = plsc.load_gather(adst_t, [idst])
      e = a1 + a2
      e = jnp.where(e >= 0.0, e, 0.2 * e)
      w = jnp.exp(e - mv)
      return w, isrc, idst

    def p1_chunk(k, carry):
      base = sid * T1 + k * C
      pltpu.sync_copy(src_hbm.at[pl.ds(base, C)], srcb)
      pltpu.sync_copy(dst_hbm.at[pl.ds(base, C)], dstb)
      for i in range(C // L):
        w, _, idst = edge_w(i)
        hi = lax.shift_right_logical(idst, 4)
        lo = lax.bitwise_and(idst, L - 1)
        plsc.addupdate_scatter(s_t, [hi, lo], w)
      return carry

    lax.fori_loop(0, K1, p1_chunk, 0)
    plsc.subcore_barrier()

    pltpu.sync_copy(s_t, s_sc.at[linb], add=True)
    plsc.subcore_barrier()
    pltpu.sync_copy(s_sc, s_t)

    def p2_chunk(k, carry):
      base = wid * T2 + k * C
      pltpu.sync_copy(src_hbm.at[pl.ds(base, C)], srcb)
      pltpu.sync_copy(dst_hbm.at[pl.ds(base, C)], dstb)
      pltpu.async_copy(h_hbm.at[srcb], rows, sem).wait()
      for i in range(C // L):
        w, _, idst = edge_w(i)
        hi = lax.shift_right_logical(idst, 4)
        lo = lax.bitwise_and(idst, L - 1)
        sden = plsc.load_gather(s_t, [hi, lo])
        coef = w / (sden + 1e-16)
        for j in range(L):
          r = i * L + j
          cv = coef.at[jnp.full((L,), j, jnp.int32)].get(
              mode="promise_in_bounds")
          for q in range(D // L):
            rows[r, pl.ds(q * L, L)] = rows[r, pl.ds(q * L, L)] * cv

      pltpu.sync_copy(rows, out_sc.at[dstb], add=True)
      return carry

    lax.fori_loop(0, K2, p2_chunk, 0)
    plsc.subcore_barrier()

    pltpu.sync_copy(out_sc.at[pl.ds(sid * RPT, RPT)],
                    out_hbm.at[cid, pl.ds(sid * RPT, RPT)])

  return gat


def kernel(x, edge_index,
           phy1_W, phy1_b, phy2_W, phy2_b, poi1_W, poi1_b, poi2_W, poi2_b,
           se1_W, se1_b, se2_W, se2_b, scene1_W, scene1_b, scene2_W, scene2_b,
           fc1_W, fc1_b, fc2_W, fc2_b, fc3_W, fc3_b,
           conv1_W, conv1_as, conv1_ad, conv1_b,
           conv2_W, conv2_as, conv2_ad, conv2_b):
  n = x.shape[0]
  e = edge_index.shape[1]
  ea = e + n
  epad = ((ea + NC * NS * C - 1) // (NC * NS * C)) * (NC * NS * C)

  xpad = jnp.zeros((NPAD, x.shape[1]), _f32).at[:n].set(x)
  xphy = xpad[:, 0:2]
  xpoi = xpad[:, 2:15]
  xse = xpad[:, 15:55]
  xsc = xpad[:, 55:]

  wdict = dict(
      p1w=phy1_W, p1b=phy1_b.reshape(1, -1),
      p2w=phy2_W, p2b=phy2_b.reshape(1, -1),
      o1w=poi1_W, o1b=poi1_b.reshape(1, -1),
      o2w=poi2_W, o2b=poi2_b.reshape(1, -1),
      s1w=se1_W, s1b=se1_b.reshape(1, -1),
      s2w=se2_W, s2b=se2_b.reshape(1, -1),
      c1w=scene1_W, c1b=scene1_b.reshape(1, -1),
      c2w=scene2_W, c2b=scene2_b.reshape(1, -1),
      wp=conv1_W[0:64], wo=conv1_W[64:128], ws=conv1_W[128:256],
      wc=conv1_W[256:320],
      avs=conv1_as.reshape(1, -1), avd=conv1_ad.reshape(1, -1),
  )
  h1, a1s, a1d, ms1, md1 = _encoder_call(xphy, xpoi, xse, xsc, wdict)

  loop = jnp.arange(n, dtype=edge_index.dtype)
  padv = jnp.full((epad - ea,), n, edge_index.dtype)
  src = jnp.concatenate([edge_index[0], loop, padv])
  dst = jnp.concatenate([edge_index[1], loop, padv])

  linr = jnp.arange(NPAD // L, dtype=jnp.int32)
  zs = jnp.zeros((NPAD // L, L), _f32)

  def mbound(ms, md):
    m = ms[0, 0] + md[0, 0]
    m = jnp.where(m > 0.0, m, 0.2 * m)
    return jnp.full((L,), m, _f32)

  gat1 = _make_gat_sc(128, epad)
  z128 = jnp.zeros((NPAD, 128), _f32)
  agg1 = gat1(h1, a1s.reshape(NPAD), a1d.reshape(NPAD), mbound(ms1, md1),
              src, dst, linr, zs, z128)

  h2, a2s, a2d, ms2, md2 = _mid_call(
      agg1[0], agg1[1], conv1_b.reshape(1, -1), conv2_W,
      conv2_as.reshape(1, -1), conv2_ad.reshape(1, -1))

  gat2 = _make_gat_sc(64, epad)
  z64 = jnp.zeros((NPAD, 64), _f32)
  agg2 = gat2(h2, a2s.reshape(NPAD), a2d.reshape(NPAD), mbound(ms2, md2),
              src, dst, linr, zs, z64)

  out = _head_call(agg2[0], agg2[1], conv2_b.reshape(1, -1),
                   fc1_W, fc1_b.reshape(1, -1), fc2_W, fc2_b.reshape(1, -1),
                   fc3_W, fc3_b.reshape(1, -1))
  return out[:n]


# speedup vs baseline: 31.2496x; 1.9817x over previous
import functools

import jax
import jax.numpy as jnp
from jax import lax
from jax.experimental import pallas as pl
from jax.experimental.pallas import tpu as pltpu
from jax.experimental.pallas import tpu_sc as plsc

N = 10000
NPAD = 10240
RB = 640
GRID = NPAD // RB
C = 128
NC, NS, L = 2, 16, 16

_f32 = jnp.float32


def _full_spec(shape):
  nd = len(shape)
  return pl.BlockSpec(shape, lambda i: (0,) * nd)


def _row_spec(d):
  return pl.BlockSpec((RB, d), lambda i: (i, 0))


def _enc_body(xphy, xpoi, xse, xsc,
              p1w, p1b, p2w, p2b, o1w, o1b, o2w, o2b,
              s1w, s1b, s2w, s2b, c1w, c1b, c2w, c2b,
              wp, wo, ws, wc, avs, avd,
              h1_ref, asrc_ref, adst_ref, ms_ref, md_ref):
  relu = lambda v: jnp.maximum(v, 0.0)
  xp = relu(relu(xphy[...] @ p1w[...] + p1b[...]) @ p2w[...] + p2b[...])
  xo = relu(relu(xpoi[...] @ o1w[...] + o1b[...]) @ o2w[...] + o2b[...])
  xs = relu(relu(xse[...] @ s1w[...] + s1b[...]) @ s2w[...] + s2b[...])
  z = xsc[...] @ c1w[...] + c1b[...]
  sg = 1.0 / (1.0 + jnp.exp(-z))
  xc = relu(sg @ c2w[...] + c2b[...])
  h1 = xp @ wp[...] + xo @ wo[...] + xs @ ws[...] + xc @ wc[...]
  h1_ref[0] = h1[:, 0:64]
  h1_ref[1] = h1[:, 64:128]
  a_s = jnp.sum(h1 * avs[...], axis=1, keepdims=True)
  a_d = jnp.sum(h1 * avd[...], axis=1, keepdims=True)
  asrc_ref[...] = a_s
  adst_ref[...] = a_d
  bs = jnp.max(a_s, axis=(0, 1), keepdims=True)
  bd = jnp.max(a_d, axis=(0, 1), keepdims=True)
  first = pl.program_id(0) == 0

  @pl.when(first)
  def _():
    ms_ref[...] = bs
    md_ref[...] = bd

  @pl.when(jnp.logical_not(first))
  def _():
    ms_ref[...] = jnp.maximum(ms_ref[...], bs)
    md_ref[...] = jnp.maximum(md_ref[...], bd)


def _encoder_call(xphy, xpoi, xse, xsc, wdict):
  in_specs = [_row_spec(2), _row_spec(13), _row_spec(40), _row_spec(365)]
  ws = [wdict[k] for k in ('p1w', 'p1b', 'p2w', 'p2b', 'o1w', 'o1b', 'o2w',
                           'o2b', 's1w', 's1b', 's2w', 's2b', 'c1w', 'c1b',
                           'c2w', 'c2b', 'wp', 'wo', 'ws', 'wc', 'avs', 'avd')]
  in_specs += [_full_spec(w.shape) for w in ws]
  out_shapes = (
      jax.ShapeDtypeStruct((2, NPAD, 64), _f32),
      jax.ShapeDtypeStruct((NPAD, 1), _f32),
      jax.ShapeDtypeStruct((NPAD, 1), _f32),
      jax.ShapeDtypeStruct((1, 1), _f32),
      jax.ShapeDtypeStruct((1, 1), _f32),
  )
  out_specs = (
      pl.BlockSpec((2, RB, 64), lambda i: (0, i, 0)),
      _row_spec(1), _row_spec(1),
      pl.BlockSpec((1, 1), lambda i: (0, 0)),
      pl.BlockSpec((1, 1), lambda i: (0, 0)),
  )
  return pl.pallas_call(
      _enc_body, grid=(GRID,), in_specs=in_specs, out_specs=out_specs,
      out_shape=out_shapes)(xphy, xpoi, xse, xsc, *ws)


def _mid_body(agg, s0, b1, w2, avs, avd,
              h2_ref, asrc_ref, adst_ref, ms_ref, md_ref):
  a = agg[...]
  den = s0[...] + 1e-16
  cc = jnp.concatenate([a[0], a[1]], axis=1)
  g = jnp.maximum(cc / den + b1[...], 0.0)
  h2 = g @ w2[...]
  h2_ref[...] = h2
  a_s = jnp.sum(h2 * avs[...], axis=1, keepdims=True)
  a_d = jnp.sum(h2 * avd[...], axis=1, keepdims=True)
  asrc_ref[...] = a_s
  adst_ref[...] = a_d
  bs = jnp.max(a_s, axis=(0, 1), keepdims=True)
  bd = jnp.max(a_d, axis=(0, 1), keepdims=True)
  first = pl.program_id(0) == 0

  @pl.when(first)
  def _():
    ms_ref[...] = bs
    md_ref[...] = bd

  @pl.when(jnp.logical_not(first))
  def _():
    ms_ref[...] = jnp.maximum(ms_ref[...], bs)
    md_ref[...] = jnp.maximum(md_ref[...], bd)


def _mid_call(agg, s0, b1, w2, avs, avd):
  in_specs = [pl.BlockSpec((2, RB, 64), lambda i: (0, i, 0)),
              _row_spec(1), _full_spec(b1.shape),
              _full_spec(w2.shape), _full_spec(avs.shape),
              _full_spec(avd.shape)]
  out_shapes = (
      jax.ShapeDtypeStruct((NPAD, 64), _f32),
      jax.ShapeDtypeStruct((NPAD, 1), _f32),
      jax.ShapeDtypeStruct((NPAD, 1), _f32),
      jax.ShapeDtypeStruct((1, 1), _f32),
      jax.ShapeDtypeStruct((1, 1), _f32),
  )
  out_specs = (
      _row_spec(64), _row_spec(1), _row_spec(1),
      pl.BlockSpec((1, 1), lambda i: (0, 0)),
      pl.BlockSpec((1, 1), lambda i: (0, 0)),
  )
  return pl.pallas_call(
      _mid_body, grid=(GRID,), in_specs=in_specs, out_specs=out_specs,
      out_shape=out_shapes)(agg, s0, b1, w2, avs, avd)


def _head_body(agg, s0, s1, b2, f1w, f1b, f2w, f2b, f3w, f3b, out_ref):
  relu = lambda v: jnp.maximum(v, 0.0)
  a = agg[...]
  den = s0[...] + s1[...] + 1e-16
  g = relu((a[0] + a[1]) / den + b2[...])
  y = relu(g @ f1w[...] + f1b[...])
  y = relu(y @ f2w[...] + f2b[...])
  out_ref[...] = y @ f3w[...] + f3b[...]


def _head_call(agg, s0, s1, b2, f1w, f1b, f2w, f2b, f3w, f3b):
  ws = [b2, f1w, f1b, f2w, f2b, f3w, f3b]
  in_specs = ([pl.BlockSpec((2, RB, 64), lambda i: (0, i, 0)),
               _row_spec(1), _row_spec(1)] +
              [_full_spec(w.shape) for w in ws])
  return pl.pallas_call(
      _head_body, grid=(GRID,), in_specs=in_specs,
      out_specs=_row_spec(24),
      out_shape=jax.ShapeDtypeStruct((NPAD, 24), _f32))(agg, s0, s1, *ws)


def _make_gat_sc_feat(Dh, EPAD):
  T = EPAD // NS
  K = T // C
  IB = 27
  NBLK = K // IB
  NR = NPAD // L
  RPT = NPAD // NS
  SPT = NR // NS

  mesh = plsc.VectorSubcoreMesh(core_axis_name="c", subcore_axis_name="s",
                                num_cores=NC, num_subcores=NS)

  @functools.partial(
      pl.kernel, mesh=mesh,
      compiler_params=pltpu.CompilerParams(needs_layout_passes=False,
                                           use_tc_tiling_on_sc=False),
      out_type=(jax.ShapeDtypeStruct((NC, NPAD, Dh), _f32),
                jax.ShapeDtypeStruct((NC, NR, L), _f32)),
      scratch_types=dict(
          asrc_t=pltpu.VMEM((NPAD,), _f32),
          adst_t=pltpu.VMEM((NPAD,), _f32),
          s_t=pltpu.VMEM((NR, L), _f32),
          mvec_t=pltpu.VMEM((L,), _f32),
          srcs=pltpu.VMEM((IB, C), jnp.int32),
          dsts=pltpu.VMEM((IB, C), jnp.int32),
          linb=pltpu.VMEM((NR,), jnp.int32),
          rows0=pltpu.VMEM((C, Dh), _f32),
          rows1=pltpu.VMEM((C, Dh), _f32),
          s_sc=pltpu.VMEM_SHARED((NR, L), _f32),
          out_sc=pltpu.VMEM_SHARED((NPAD, Dh), _f32),
          sem0=pltpu.SemaphoreType.DMA,
          sem1=pltpu.SemaphoreType.DMA,
      ),
  )
  def gat(h_hbm, asrc_hbm, adst_hbm, mvec_hbm, src_hbm, dst_hbm, linr_hbm,
          zs_hbm, out_hbm, outs_hbm,
          asrc_t, adst_t, s_t, mvec_t, srcs, dsts, linb, rows0, rows1,
          s_sc, out_sc, sem0, sem1):
    cid = lax.axis_index("c")
    sid = lax.axis_index("s")
    rows = (rows0, rows1)
    sems = (sem0, sem1)

    pltpu.sync_copy(asrc_hbm, asrc_t)
    pltpu.sync_copy(adst_hbm, adst_t)
    pltpu.sync_copy(mvec_hbm, mvec_t)
    pltpu.sync_copy(linr_hbm, linb)
    pltpu.sync_copy(zs_hbm, s_t)

    def zrow(r, carry):
      for q in range(Dh // L):
        rows0[r, pl.ds(q * L, L)] = jnp.zeros((L,), _f32)
      return carry

    lax.fori_loop(0, C, zrow, 0)
    for t in range(RPT // C):
      pltpu.sync_copy(rows0, out_sc.at[pl.ds(sid * RPT + t * C, C)])

    @pl.when(sid == 0)
    def _():
      pltpu.sync_copy(zs_hbm, s_sc)

    plsc.subcore_barrier()

    mv = mvec_t[...]
    offv = jnp.full((L,), cid * NPAD, jnp.int32)

    def issue_gather(k, b):
      return pltpu.async_copy(h_hbm.at[srcs.at[k]], rows[b], sems[b])

    def process(k, b):
      coefs = []
      for i in range(C // L):
        isrc = srcs[k, pl.ds(i * L, L)] - offv
        idst = dsts[k, pl.ds(i * L, L)]
        a1 = plsc.load_gather(asrc_t, [isrc])
        a2 = plsc.load_gather(adst_t, [idst])
        e = a1 + a2
        e = jnp.where(e >= 0.0, e, 0.2 * e)
        w = jnp.exp(e - mv)
        hi = lax.shift_right_logical(idst, 4)
        lo = lax.bitwise_and(idst, L - 1)
        plsc.addupdate_scatter(s_t, [hi, lo], w)
        coefs.append(w)
      pltpu.make_async_copy(h_hbm.at[srcs.at[k]], rows[b], sems[b]).wait()
      rb = rows[b]
      for i in range(C // L):
        w = coefs[i]
        for j in range(L):
          r = i * L + j
          cv = w.at[jnp.full((L,), j, jnp.int32)].get(
              mode="promise_in_bounds")
          for q in range(Dh // L):
            rb[r, pl.ds(q * L, L)] = rb[r, pl.ds(q * L, L)] * cv
      pltpu.sync_copy(rb, out_sc.at[dsts.at[k]], add=True)

    def block(blk, carry):
      base = sid * K + blk * IB
      pltpu.sync_copy(src_hbm.at[pl.ds(base, IB)], srcs)
      pltpu.sync_copy(dst_hbm.at[pl.ds(base, IB)], dsts)

      def orow(r, c2):
        for q in range(C // L):
          srcs[r, pl.ds(q * L, L)] = srcs[r, pl.ds(q * L, L)] + offv
        return c2

      lax.fori_loop(0, IB, orow, 0)

      issue_gather(0, 0)

      def pair(kk, c2):
        k0 = kk * 2
        issue_gather(k0 + 1, 1)
        process(k0, 0)
        issue_gather(k0 + 2, 0)
        process(k0 + 1, 1)
        return c2

      lax.fori_loop(0, (IB - 1) // 2, pair, 0)
      process(IB - 1, (IB - 1) % 2)
      return carry

    lax.fori_loop(0, NBLK, block, 0)

    pltpu.sync_copy(s_t, s_sc.at[linb], add=True)
    plsc.subcore_barrier()

    pltpu.sync_copy(out_sc.at[pl.ds(sid * RPT, RPT)],
                    out_hbm.at[cid, pl.ds(sid * RPT, RPT)])
    pltpu.sync_copy(s_sc.at[pl.ds(sid * SPT, SPT)],
                    outs_hbm.at[cid, pl.ds(sid * SPT, SPT)])

  return gat


def _make_gat_sc(D, EPAD):
  T2 = EPAD // (NC * NS)
  K2 = T2 // C
  NR = NPAD // L
  RPT = NPAD // NS
  SPT = NR // NS

  mesh = plsc.VectorSubcoreMesh(core_axis_name="c", subcore_axis_name="s",
                                num_cores=NC, num_subcores=NS)

  @functools.partial(
      pl.kernel, mesh=mesh,
      compiler_params=pltpu.CompilerParams(needs_layout_passes=False,
                                           use_tc_tiling_on_sc=False),
      out_type=(jax.ShapeDtypeStruct((NC, NPAD, D), _f32),
                jax.ShapeDtypeStruct((NC, NR, L), _f32)),
      scratch_types=dict(
          asrc_t=pltpu.VMEM((NPAD,), _f32),
          adst_t=pltpu.VMEM((NPAD,), _f32),
          s_t=pltpu.VMEM((NR, L), _f32),
          mvec_t=pltpu.VMEM((L,), _f32),
          srcs=pltpu.VMEM((K2, C), jnp.int32),
          dsts=pltpu.VMEM((K2, C), jnp.int32),
          linb=pltpu.VMEM((NR,), jnp.int32),
          rows0=pltpu.VMEM((C, D), _f32),
          rows1=pltpu.VMEM((C, D), _f32),
          s_sc=pltpu.VMEM_SHARED((NR, L), _f32),
          out_sc=pltpu.VMEM_SHARED((NPAD, D), _f32),
          sem0=pltpu.SemaphoreType.DMA,
          sem1=pltpu.SemaphoreType.DMA,
      ),
  )
  def gat(h_hbm, asrc_hbm, adst_hbm, mvec_hbm, src_hbm, dst_hbm, linr_hbm,
          zs_hbm, out_hbm, outs_hbm,
          asrc_t, adst_t, s_t, mvec_t, srcs, dsts, linb, rows0, rows1,
          s_sc, out_sc, sem0, sem1):
    cid = lax.axis_index("c")
    sid = lax.axis_index("s")
    wid = sid * NC + cid
    rows = (rows0, rows1)
    sems = (sem0, sem1)

    pltpu.sync_copy(asrc_hbm, asrc_t)
    pltpu.sync_copy(adst_hbm, adst_t)
    pltpu.sync_copy(mvec_hbm, mvec_t)
    pltpu.sync_copy(linr_hbm, linb)
    pltpu.sync_copy(zs_hbm, s_t)
    pltpu.sync_copy(src_hbm.at[pl.ds(wid * K2, K2)], srcs)
    pltpu.sync_copy(dst_hbm.at[pl.ds(wid * K2, K2)], dsts)

    def zrow(r, carry):
      for q in range(D // L):
        rows0[r, pl.ds(q * L, L)] = jnp.zeros((L,), _f32)
      return carry

    lax.fori_loop(0, C, zrow, 0)
    for t in range(RPT // C):
      pltpu.sync_copy(rows0, out_sc.at[pl.ds(sid * RPT + t * C, C)])

    @pl.when(sid == 0)
    def _():
      pltpu.sync_copy(zs_hbm, s_sc)

    plsc.subcore_barrier()

    mv = mvec_t[...]

    def issue_gather(k, b):
      return pltpu.async_copy(h_hbm.at[srcs.at[k]], rows[b], sems[b])

    def process(k, b):
      coefs = []
      for i in range(C // L):
        isrc = srcs[k, pl.ds(i * L, L)]
        idst = dsts[k, pl.ds(i * L, L)]
        a1 = plsc.load_gather(asrc_t, [isrc])
        a2 = plsc.load_gather(adst_t, [idst])
        e = a1 + a2
        e = jnp.where(e >= 0.0, e, 0.2 * e)
        w = jnp.exp(e - mv)
        hi = lax.shift_right_logical(idst, 4)
        lo = lax.bitwise_and(idst, L - 1)
        plsc.addupdate_scatter(s_t, [hi, lo], w)
        coefs.append(w)
      pltpu.make_async_copy(h_hbm.at[srcs.at[k]], rows[b], sems[b]).wait()
      rb = rows[b]
      for i in range(C // L):
        w = coefs[i]
        for j in range(L):
          r = i * L + j
          cv = w.at[jnp.full((L,), j, jnp.int32)].get(
              mode="promise_in_bounds")
          for q in range(D // L):
            rb[r, pl.ds(q * L, L)] = rb[r, pl.ds(q * L, L)] * cv
      pltpu.sync_copy(rb, out_sc.at[dsts.at[k]], add=True)

    issue_gather(0, 0)

    def pair(kk, carry):
      k0 = kk * 2
      issue_gather(k0 + 1, 1)
      process(k0, 0)
      issue_gather(k0 + 2, 0)
      process(k0 + 1, 1)
      return carry

    lax.fori_loop(0, (K2 - 1) // 2, pair, 0)
    process(K2 - 1, (K2 - 1) % 2)

    pltpu.sync_copy(s_t, s_sc.at[linb], add=True)
    plsc.subcore_barrier()

    pltpu.sync_copy(out_sc.at[pl.ds(sid * RPT, RPT)],
                    out_hbm.at[cid, pl.ds(sid * RPT, RPT)])
    pltpu.sync_copy(s_sc.at[pl.ds(sid * SPT, SPT)],
                    outs_hbm.at[cid, pl.ds(sid * SPT, SPT)])

  return gat


def kernel(x, edge_index,
           phy1_W, phy1_b, phy2_W, phy2_b, poi1_W, poi1_b, poi2_W, poi2_b,
           se1_W, se1_b, se2_W, se2_b, scene1_W, scene1_b, scene2_W, scene2_b,
           fc1_W, fc1_b, fc2_W, fc2_b, fc3_W, fc3_b,
           conv1_W, conv1_as, conv1_ad, conv1_b,
           conv2_W, conv2_as, conv2_ad, conv2_b):
  n = x.shape[0]
  e = edge_index.shape[1]
  ea = e + n
  epad = ((ea + NC * NS * C - 1) // (NC * NS * C)) * (NC * NS * C)

  xpad = jnp.zeros((NPAD, x.shape[1]), _f32).at[:n].set(x)
  xphy = xpad[:, 0:2]
  xpoi = xpad[:, 2:15]
  xse = xpad[:, 15:55]
  xsc = xpad[:, 55:]

  wdict = dict(
      p1w=phy1_W, p1b=phy1_b.reshape(1, -1),
      p2w=phy2_W, p2b=phy2_b.reshape(1, -1),
      o1w=poi1_W, o1b=poi1_b.reshape(1, -1),
      o2w=poi2_W, o2b=poi2_b.reshape(1, -1),
      s1w=se1_W, s1b=se1_b.reshape(1, -1),
      s2w=se2_W, s2b=se2_b.reshape(1, -1),
      c1w=scene1_W, c1b=scene1_b.reshape(1, -1),
      c2w=scene2_W, c2b=scene2_b.reshape(1, -1),
      wp=conv1_W[0:64], wo=conv1_W[64:128], ws=conv1_W[128:256],
      wc=conv1_W[256:320],
      avs=conv1_as.reshape(1, -1), avd=conv1_ad.reshape(1, -1),
  )
  h1, a1s, a1d, ms1, md1 = _encoder_call(xphy, xpoi, xse, xsc, wdict)

  loop = jnp.arange(n, dtype=edge_index.dtype)
  padv = jnp.full((epad - ea,), n, edge_index.dtype)
  src2 = jnp.concatenate([edge_index[0], loop, padv]).reshape(epad // C, C)
  dst2 = jnp.concatenate([edge_index[1], loop, padv]).reshape(epad // C, C)

  linr = jnp.arange(NPAD // L, dtype=jnp.int32)
  zs = jnp.zeros((NPAD // L, L), _f32)

  def mbound(ms, md):
    m = ms[0, 0] + md[0, 0]
    m = jnp.where(m > 0.0, m, 0.2 * m)
    return jnp.full((L,), m, _f32)

  gat1 = _make_gat_sc_feat(64, epad)
  agg1, s1p = gat1(h1.reshape(2 * NPAD, 64), a1s.reshape(NPAD),
                   a1d.reshape(NPAD), mbound(ms1, md1), src2, dst2, linr, zs)

  h2, a2s, a2d, ms2, md2 = _mid_call(
      agg1, s1p[0].reshape(NPAD, 1),
      conv1_b.reshape(1, -1), conv2_W,
      conv2_as.reshape(1, -1), conv2_ad.reshape(1, -1))

  gat2 = _make_gat_sc(64, epad)
  agg2, s2p = gat2(h2, a2s.reshape(NPAD), a2d.reshape(NPAD), mbound(ms2, md2),
                   src2, dst2, linr, zs)

  out = _head_call(agg2, s2p[0].reshape(NPAD, 1), s2p[1].reshape(NPAD, 1),
                   conv2_b.reshape(1, -1),
                   fc1_W, fc1_b.reshape(1, -1), fc2_W, fc2_b.reshape(1, -1),
                   fc3_W, fc3_b.reshape(1, -1))
  return out[:n]
